# Initial kernel scaffold; baseline (speedup 1.0000x reference)
import jax
import jax.numpy as jnp
from jax.experimental import pallas as pl


def kernel(x, weight):
    raise NotImplementedError("write your pallas kernel here")


# speedup vs baseline: 1.5064x; 1.5064x over previous
import jax
import jax.numpy as jnp
from jax.experimental import pallas as pl
from jax.experimental.pallas import tpu as pltpu

N_EXP = 8
TOKEN_BLOCK = 2048


def _gate_body(x_ref, wt_ref, w_out_ref, i_out_ref):
    x = x_ref[...]
    wt = wt_ref[...]
    scores = jax.lax.dot_general(
        x, wt, (((1,), (0,)), ((), ())),
        preferred_element_type=jnp.float32)
    m = jnp.max(scores, axis=1, keepdims=True)
    e = jnp.exp(scores - m)
    p = e / jnp.sum(e, axis=1, keepdims=True)

    ii = jax.lax.broadcasted_iota(jnp.int32, p.shape, 1)
    m1 = jnp.max(p, axis=1, keepdims=True)
    idx1 = jnp.min(jnp.where(p == m1, ii, N_EXP), axis=1, keepdims=True)
    p2 = jnp.where(ii == idx1, -1.0, p)
    m2 = jnp.max(p2, axis=1, keepdims=True)
    idx2 = jnp.min(jnp.where(p2 == m2, ii, N_EXP), axis=1, keepdims=True)

    w_out_ref[...] = jnp.concatenate([m1, m2], axis=1)
    i_out_ref[...] = jnp.concatenate([idx1, idx2], axis=1)


def kernel(x, weight):
    n_tokens, dim = x.shape
    grid = (n_tokens // TOKEN_BLOCK,)
    weights, indices = pl.pallas_call(
        _gate_body,
        grid=grid,
        in_specs=[
            pl.BlockSpec((TOKEN_BLOCK, dim), lambda i: (i, 0)),
            pl.BlockSpec((dim, N_EXP), lambda i: (0, 0)),
        ],
        out_specs=[
            pl.BlockSpec((TOKEN_BLOCK, 2), lambda i: (i, 0)),
            pl.BlockSpec((TOKEN_BLOCK, 2), lambda i: (i, 0)),
        ],
        out_shape=[
            jax.ShapeDtypeStruct((n_tokens, 2), jnp.float32),
            jax.ShapeDtypeStruct((n_tokens, 2), jnp.int32),
        ],
        compiler_params=pltpu.CompilerParams(
            dimension_semantics=("arbitrary",),
        ),
    )(x, weight.T)
    return weights, indices
